# Initial kernel scaffold; baseline (speedup 1.0000x reference)
import jax
import jax.numpy as jnp
from jax.experimental import pallas as pl


def kernel(x, batch, size, Wg, bg):
    raise NotImplementedError("write your pallas kernel here")


# speedup vs baseline: 4.9728x; 4.9728x over previous
import functools

import jax
import jax.numpy as jnp
from jax.experimental import pallas as pl
from jax.experimental.pallas import tpu as pltpu

N, D, G = 100000, 128, 512
B = 2000
NB = N // B

_NEG = -1e30


def _onehot_mask(b, n_rows):
    return b[:, None] == jax.lax.broadcasted_iota(jnp.int32, (n_rows, G), 1)


def _stage_a_kernel(x_ref, b3_ref, wg_ref, bg_ref, gate_ref, segmax_ref, smax_acc):
    i = pl.program_id(0)

    @pl.when(i == 0)
    def _():
        smax_acc[...] = jnp.full((1, G), _NEG, jnp.float32)

    x = x_ref[...]
    w = wg_ref[...][:, 0]
    gate = jnp.sum(x * w[None, :], axis=1, keepdims=True) + bg_ref[0, 0]
    gate_ref[...] = gate.reshape(1, 1, B)

    mask = _onehot_mask(b3_ref[0, 0, :], B)
    masked = jnp.where(mask, gate, _NEG)
    smax_acc[...] = jnp.maximum(smax_acc[...],
                                jnp.max(masked, axis=0, keepdims=True))

    @pl.when(i == NB - 1)
    def _():
        segmax_ref[...] = smax_acc[...].reshape(G, 1)


def _stage_b_kernel(gate_ref, b3_ref, segmax_ref, e_ref, recip_ref, den_acc):
    i = pl.program_id(0)

    @pl.when(i == 0)
    def _():
        den_acc[...] = jnp.zeros((G, 1), jnp.float32)

    mask = _onehot_mask(b3_ref[0, 0, :], B)
    maskf = mask.astype(jnp.float32)
    max_row = jax.lax.dot_general(
        maskf, segmax_ref[...],
        (((1,), (0,)), ((), ())),
        preferred_element_type=jnp.float32)
    gate = gate_ref[...].reshape(B, 1)
    e = jnp.exp(gate - max_row)
    e_ref[...] = e.reshape(1, 1, B)
    den_acc[...] += jax.lax.dot_general(
        maskf, e,
        (((0,), (0,)), ((), ())),
        preferred_element_type=jnp.float32)

    @pl.when(i == NB - 1)
    def _():
        recip_ref[...] = 1.0 / (den_acc[...] + 1e-16)


def _stage_c_kernel(x_ref, e3_ref, b3_ref, recip_ref, g_ref, out_ref, out_acc):
    i = pl.program_id(0)

    @pl.when(i == 0)
    def _():
        out_acc[...] = jnp.zeros((G, D), jnp.float32)

    mask = _onehot_mask(b3_ref[0, 0, :], B)
    maskf = mask.astype(jnp.float32)
    r_row = jax.lax.dot_general(
        maskf, recip_ref[...],
        (((1,), (0,)), ((), ())),
        preferred_element_type=jnp.float32)
    g = e3_ref[...].reshape(B, 1) * r_row
    g_ref[...] = g.reshape(1, 1, B)

    vals = x_ref[...] * g
    out_acc[...] += jax.lax.dot_general(
        mask.astype(jnp.bfloat16), vals.astype(jnp.bfloat16),
        (((0,), (0,)), ((), ())),
        preferred_element_type=jnp.float32)

    @pl.when(i == NB - 1)
    def _():
        out_ref[...] = out_acc[...]


def kernel(x, batch, size, Wg, bg):
    del size
    b3 = batch.astype(jnp.int32).reshape(NB, 1, B)
    bg2 = bg.reshape(1, 1)

    gate3, segmax = pl.pallas_call(
        _stage_a_kernel,
        grid=(NB,),
        in_specs=[
            pl.BlockSpec((B, D), lambda i: (i, 0)),
            pl.BlockSpec((1, 1, B), lambda i: (i, 0, 0)),
            pl.BlockSpec((D, 1), lambda i: (0, 0)),
            pl.BlockSpec((1, 1), lambda i: (0, 0)),
        ],
        out_specs=[
            pl.BlockSpec((1, 1, B), lambda i: (i, 0, 0)),
            pl.BlockSpec((G, 1), lambda i: (0, 0)),
        ],
        out_shape=[
            jax.ShapeDtypeStruct((NB, 1, B), jnp.float32),
            jax.ShapeDtypeStruct((G, 1), jnp.float32),
        ],
        scratch_shapes=[pltpu.VMEM((1, G), jnp.float32)],
    )(x, b3, Wg, bg2)

    e3, recip = pl.pallas_call(
        _stage_b_kernel,
        grid=(NB,),
        in_specs=[
            pl.BlockSpec((1, 1, B), lambda i: (i, 0, 0)),
            pl.BlockSpec((1, 1, B), lambda i: (i, 0, 0)),
            pl.BlockSpec((G, 1), lambda i: (0, 0)),
        ],
        out_specs=[
            pl.BlockSpec((1, 1, B), lambda i: (i, 0, 0)),
            pl.BlockSpec((G, 1), lambda i: (0, 0)),
        ],
        out_shape=[
            jax.ShapeDtypeStruct((NB, 1, B), jnp.float32),
            jax.ShapeDtypeStruct((G, 1), jnp.float32),
        ],
        scratch_shapes=[pltpu.VMEM((G, 1), jnp.float32)],
    )(gate3, b3, segmax)

    g3, out = pl.pallas_call(
        _stage_c_kernel,
        grid=(NB,),
        in_specs=[
            pl.BlockSpec((B, D), lambda i: (i, 0)),
            pl.BlockSpec((1, 1, B), lambda i: (i, 0, 0)),
            pl.BlockSpec((1, 1, B), lambda i: (i, 0, 0)),
            pl.BlockSpec((G, 1), lambda i: (0, 0)),
        ],
        out_specs=[
            pl.BlockSpec((1, 1, B), lambda i: (i, 0, 0)),
            pl.BlockSpec((G, D), lambda i: (0, 0)),
        ],
        out_shape=[
            jax.ShapeDtypeStruct((NB, 1, B), jnp.float32),
            jax.ShapeDtypeStruct((G, D), jnp.float32),
        ],
        scratch_shapes=[pltpu.VMEM((G, D), jnp.float32)],
    )(x, e3, b3, recip)

    g = g3.reshape(N, 1)
    return (out, g)
